# baseline (speedup 1.0000x reference)
import jax
import jax.numpy as jnp
from jax import lax
from jax.experimental import pallas as pl
from jax.experimental.pallas import tpu as pltpu
from jax.experimental.pallas import tpu_sc as plsc

VOCAB = 1000000
DIM = 32
B, L = 16384, 26

NC, NS = 2, 16
NW = NC * NS
BPW = B // NW
CHUNK = 128
NCHUNKB = BPW // CHUNK
NSTEP = L * NCHUNKB
TROWS = VOCAB * DIM // 128


def _body(tab, idx2, out_t, idxv, rowidx, colbase, rows, staging, sem):
    wid = lax.axis_index("s") * NC + lax.axis_index("c")
    iota16 = lax.iota(jnp.int32, 16)

    def step(t, carry):
        l = t % L
        c = t // L
        r = l * 128 + wid * NCHUNKB + c
        b0 = wid * BPW + c * CHUNK
        pltpu.sync_copy(idx2.at[r], idxv)
        for i0 in range(0, CHUNK, 16):
            v = idxv[pl.ds(i0, 16)]
            rowidx[pl.ds(i0, 16)] = lax.shift_right_logical(v, 2)
            colbase[pl.ds(i0, 16)] = (v & 3) * DIM
        pltpu.async_copy(tab.at[rowidx], rows, sem).wait()
        for i0 in range(0, CHUNK, 16):
            iv = iota16 + i0
            cb = colbase[pl.ds(i0, 16)]
            for d in range(DIM):
                staging[d, pl.ds(i0, 16)] = plsc.load_gather(rows, [iv, cb + d])
        pltpu.sync_copy(staging, out_t.at[l, :, pl.ds(b0, CHUNK)])
        return carry

    lax.fori_loop(0, NSTEP, step, 0)


@jax.jit
def kernel(indices, tables):
    tab128 = tables.reshape(TROWS, 128)
    idx2 = jnp.transpose(indices, (1, 0)).reshape(B * L // 128, 128)
    mesh = plsc.VectorSubcoreMesh(core_axis_name="c", subcore_axis_name="s")
    out_t = pl.kernel(
        _body,
        out_type=jax.ShapeDtypeStruct((L, DIM, B), jnp.float32),
        mesh=mesh,
        scratch_types=[
            pltpu.VMEM((CHUNK,), jnp.int32),
            pltpu.VMEM((CHUNK,), jnp.int32),
            pltpu.VMEM((CHUNK,), jnp.int32),
            pltpu.VMEM((CHUNK, 128), jnp.float32),
            pltpu.VMEM((DIM, CHUNK), jnp.float32),
            pltpu.SemaphoreType.DMA,
        ],
        compiler_params=pltpu.CompilerParams(
            use_tc_tiling_on_sc=True, needs_layout_passes=False
        ),
    )(tab128, idx2)
    return jnp.transpose(out_t, (2, 0, 1))


# speedup vs baseline: 2.2478x; 2.2478x over previous
import jax
import jax.numpy as jnp
from jax import lax
from jax.experimental import pallas as pl
from jax.experimental.pallas import tpu as pltpu
from jax.experimental.pallas import tpu_sc as plsc

VOCAB = 1000000
FACTOR = 32
SPLIT = VOCAB // FACTOR
DIM = 32
B, L = 16384, 26

NC, NS = 2, 16
NW = NC * NS
BPW = B // NW
CHUNK = 128
NCHUNKB = BPW // CHUNK
NSTEP = L * NCHUNKB
VT = 244
VTAIL = VT * 128
RPS = 7816
SROWS = FACTOR * RPS


def _repack_body(tab_t, tail_block, scratch, src, staging, tailv, sem):
    s = lax.axis_index("s") * NC + lax.axis_index("c")
    iota16 = lax.iota(jnp.int32, 16)
    qrow = lax.shift_right_logical(iota16, 2)
    qcol = (iota16 & 3) * DIM

    def tile_step(t, carry):
        pltpu.sync_copy(tab_t.at[s, :, pl.ds(t * 128, 128)], src)
        for d in range(DIM):
            for vv in range(0, 128, 16):
                x = src[d, pl.ds(vv, 16)]
                rvec = qrow + (vv // 4)
                cvec = qcol + d
                plsc.store_scatter(staging, [rvec, cvec], x)
        pltpu.sync_copy(staging, scratch.at[pl.ds(s * RPS + t * 32, 32)])
        return carry

    lax.fori_loop(0, VT, tile_step, 0)
    pltpu.sync_copy(tail_block.at[s], tailv)
    pltpu.sync_copy(tailv, scratch.at[pl.ds(s * RPS + VT * 32, 8)])


def _gather_body(scratch, idx2, out_t, idxv, rowidx, colbase, rows, staging, sem):
    wid = lax.axis_index("s") * NC + lax.axis_index("c")
    iota16 = lax.iota(jnp.int32, 16)

    def step(t, carry):
        l = t % L
        c = t // L
        r = l * 128 + wid * NCHUNKB + c
        b0 = wid * BPW + c * CHUNK
        pltpu.sync_copy(idx2.at[r], idxv)
        for i0 in range(0, CHUNK, 16):
            idx = idxv[pl.ds(i0, 16)]
            s = idx // SPLIT
            v = idx - s * SPLIT
            rowidx[pl.ds(i0, 16)] = s * RPS + lax.shift_right_logical(v, 2)
            colbase[pl.ds(i0, 16)] = (v & 3) * DIM
        pltpu.async_copy(scratch.at[rowidx], rows, sem).wait()
        for i0 in range(0, CHUNK, 16):
            iv = iota16 + i0
            cb = colbase[pl.ds(i0, 16)]
            for d in range(DIM):
                staging[d, pl.ds(i0, 16)] = plsc.load_gather(rows, [iv, cb + d])
        pltpu.sync_copy(staging, out_t.at[l, :, pl.ds(b0, CHUNK)])
        return carry

    lax.fori_loop(0, NSTEP, step, 0)


@jax.jit
def kernel(indices, tables):
    tab_t = jnp.transpose(tables, (0, 2, 1))
    idx2 = jnp.transpose(indices, (1, 0)).reshape(B * L // 128, 128)
    tail = tables[:, VTAIL:SPLIT, :]
    tail_a = tail[:, :16, :].reshape(FACTOR, 4, 128)
    tail_b = jnp.pad(tail[:, 16:, :].reshape(FACTOR, 64), ((0, 0), (0, 64)))
    tail_block = jnp.concatenate(
        [tail_a, tail_b[:, None, :], jnp.zeros((FACTOR, 3, 128), jnp.float32)],
        axis=1,
    )
    mesh = plsc.VectorSubcoreMesh(core_axis_name="c", subcore_axis_name="s")
    params = pltpu.CompilerParams(
        use_tc_tiling_on_sc=True, needs_layout_passes=False
    )
    scratch = pl.kernel(
        _repack_body,
        out_type=jax.ShapeDtypeStruct((SROWS, 128), jnp.float32),
        mesh=mesh,
        scratch_types=[
            pltpu.VMEM((DIM, 128), jnp.float32),
            pltpu.VMEM((32, 128), jnp.float32),
            pltpu.VMEM((8, 128), jnp.float32),
            pltpu.SemaphoreType.DMA,
        ],
        compiler_params=params,
    )(tab_t, tail_block)
    out_t = pl.kernel(
        _gather_body,
        out_type=jax.ShapeDtypeStruct((L, DIM, B), jnp.float32),
        mesh=mesh,
        scratch_types=[
            pltpu.VMEM((CHUNK,), jnp.int32),
            pltpu.VMEM((CHUNK,), jnp.int32),
            pltpu.VMEM((CHUNK,), jnp.int32),
            pltpu.VMEM((CHUNK, 128), jnp.float32),
            pltpu.VMEM((DIM, CHUNK), jnp.float32),
            pltpu.SemaphoreType.DMA,
        ],
        compiler_params=params,
    )(scratch, idx2)
    return jnp.transpose(out_t, (2, 0, 1))


# speedup vs baseline: 3.1107x; 1.3839x over previous
import jax
import jax.numpy as jnp
from jax import lax
from jax.experimental import pallas as pl
from jax.experimental.pallas import tpu as pltpu
from jax.experimental.pallas import tpu_sc as plsc

VOCAB = 1000000
FACTOR = 32
SPLIT = VOCAB // FACTOR
DIM = 32
B, L = 16384, 26

NC, NS = 2, 16
NW = NC * NS
BPW = B // NW
CHUNK = 128
NCHUNKB = BPW // CHUNK
NSTEP = L * NCHUNKB
VT = 244
VTAIL = VT * 128
RPS = 7816
SROWS = FACTOR * RPS
NB = 4


def _repack_body(tab_t, tail_block, scratch, *refs):
    srcs = refs[0:NB]
    stagings = refs[NB : 2 * NB]
    tailv = refs[2 * NB]
    isems = refs[2 * NB + 1 : 3 * NB + 1]
    osems = refs[3 * NB + 1 : 4 * NB + 1]
    s = lax.axis_index("s") * NC + lax.axis_index("c")
    iota16 = lax.iota(jnp.int32, 16)
    qrow = lax.shift_right_logical(iota16, 2)
    qcol = (iota16 & 3) * DIM

    def issue_in(t, b):
        pltpu.async_copy(tab_t.at[s, :, pl.ds(t * 128, 128)], srcs[b], isems[b])

    def issue_out(t, b):
        pltpu.async_copy(
            stagings[b], scratch.at[pl.ds(s * RPS + t * 32, 32)], osems[b]
        )

    def wait_in(b):
        pltpu.make_async_copy(
            tab_t.at[s, :, pl.ds(0, 128)], srcs[b], isems[b]
        ).wait()

    def wait_out(b):
        pltpu.make_async_copy(
            tab_t.at[s, :, pl.ds(0, 128)], stagings[b], osems[b]
        ).wait()

    for b in range(NB - 1):
        issue_in(b, b)

    def kbody(k, carry):
        for j in range(NB):
            t = k * NB + j

            @pl.when(t + NB - 1 < VT)
            def _():
                issue_in(t + NB - 1, (j + NB - 1) % NB)

            wait_in(j)

            @pl.when(t >= NB)
            def _():
                wait_out(j)

            def permute(g, c2):
                vv = g * 16
                rvec = qrow + g * 4
                for d in range(DIM):
                    x = srcs[j][d, pl.ds(vv, 16)]
                    plsc.store_scatter(stagings[j], [rvec, qcol + d], x)
                return c2

            lax.fori_loop(0, 8, permute, 0)
            issue_out(t, j)
        return carry

    lax.fori_loop(0, VT // NB, kbody, 0)
    for b in range(NB):
        wait_out(b)
    pltpu.sync_copy(tail_block.at[s], tailv)
    pltpu.sync_copy(tailv, scratch.at[pl.ds(s * RPS + VT * 32, 8)])


def _gather_body(scratch, idx2, out_t, *refs):
    idxvs = refs[0:NB]
    rowidxs = refs[NB : 2 * NB]
    colbases = refs[2 * NB : 3 * NB]
    rowss = refs[3 * NB : 4 * NB]
    stagings = refs[4 * NB : 5 * NB]
    qsems = refs[5 * NB : 6 * NB]
    gsems = refs[6 * NB : 7 * NB]
    osems = refs[7 * NB : 8 * NB]
    wid = lax.axis_index("s") * NC + lax.axis_index("c")
    iota16 = lax.iota(jnp.int32, 16)

    def idx_row(t):
        return (t % L) * 128 + wid * NCHUNKB + t // L

    def issue_idx(t, b):
        pltpu.async_copy(idx2.at[idx_row(t)], idxvs[b], qsems[b])

    def stage_a(t, b):
        pltpu.make_async_copy(idx2.at[0], idxvs[b], qsems[b]).wait()
        for i0 in range(0, CHUNK, 16):
            idx = idxvs[b][pl.ds(i0, 16)]
            sp = idx // SPLIT
            v = idx - sp * SPLIT
            rowidxs[b][pl.ds(i0, 16)] = sp * RPS + lax.shift_right_logical(v, 2)
            colbases[b][pl.ds(i0, 16)] = (v & 3) * DIM
        pltpu.async_copy(scratch.at[rowidxs[b]], rowss[b], gsems[b])

    def stage_b(t, b):
        pltpu.make_async_copy(
            scratch.at[pl.ds(0, CHUNK)], rowss[b], gsems[b]
        ).wait()

        @pl.when(t >= NB)
        def _():
            pltpu.make_async_copy(
                scratch.at[pl.ds(0, DIM)], stagings[b], osems[b]
            ).wait()

        def extract(g, c2):
            i0 = g * 16
            iv = iota16 + i0
            cb = colbases[b][pl.ds(i0, 16)]
            for d in range(DIM):
                stagings[b][d, pl.ds(i0, 16)] = plsc.load_gather(
                    rowss[b], [iv, cb + d]
                )
            return c2

        lax.fori_loop(0, CHUNK // 16, extract, 0)
        l = t % L
        b0 = wid * BPW + (t // L) * CHUNK
        pltpu.async_copy(
            stagings[b], out_t.at[l, :, pl.ds(b0, CHUNK)], osems[b]
        )

    for b in range(NB - 1):
        issue_idx(b, b)

    def kbody(k, carry):
        for j in range(NB):
            t = k * NB + j

            @pl.when(t + NB - 1 < NSTEP)
            def _():
                issue_idx(t + NB - 1, (j + NB - 1) % NB)

            stage_a(t, j)

            @pl.when(t >= 1)
            def _():
                stage_b(t - 1, (j + NB - 1) % NB)

        return carry

    lax.fori_loop(0, NSTEP // NB, kbody, 0)
    stage_b(NSTEP - 1, (NSTEP - 1) % NB)
    for b in range(NB):
        pltpu.make_async_copy(
            scratch.at[pl.ds(0, DIM)], stagings[b], osems[b]
        ).wait()


@jax.jit
def kernel(indices, tables):
    tab_t = jnp.transpose(tables, (0, 2, 1))
    idx2 = jnp.transpose(indices, (1, 0)).reshape(B * L // 128, 128)
    tail = tables[:, VTAIL:SPLIT, :]
    tail_a = tail[:, :16, :].reshape(FACTOR, 4, 128)
    tail_b = jnp.pad(tail[:, 16:, :].reshape(FACTOR, 64), ((0, 0), (0, 64)))
    tail_block = jnp.concatenate(
        [tail_a, tail_b[:, None, :], jnp.zeros((FACTOR, 3, 128), jnp.float32)],
        axis=1,
    )
    mesh = plsc.VectorSubcoreMesh(core_axis_name="c", subcore_axis_name="s")
    params = pltpu.CompilerParams(
        use_tc_tiling_on_sc=True, needs_layout_passes=False
    )
    scratch = pl.kernel(
        _repack_body,
        out_type=jax.ShapeDtypeStruct((SROWS, 128), jnp.float32),
        mesh=mesh,
        scratch_types=[pltpu.VMEM((DIM, 128), jnp.float32) for _ in range(NB)]
        + [pltpu.VMEM((32, 128), jnp.float32) for _ in range(NB)]
        + [pltpu.VMEM((8, 128), jnp.float32)]
        + [pltpu.SemaphoreType.DMA for _ in range(2 * NB)],
        compiler_params=params,
    )(tab_t, tail_block)
    out_t = pl.kernel(
        _gather_body,
        out_type=jax.ShapeDtypeStruct((L, DIM, B), jnp.float32),
        mesh=mesh,
        scratch_types=[pltpu.VMEM((CHUNK,), jnp.int32) for _ in range(3 * NB)]
        + [pltpu.VMEM((CHUNK, 128), jnp.float32) for _ in range(NB)]
        + [pltpu.VMEM((DIM, CHUNK), jnp.float32) for _ in range(NB)]
        + [pltpu.SemaphoreType.DMA for _ in range(3 * NB)],
        compiler_params=params,
    )(scratch, idx2)
    return jnp.transpose(out_t, (2, 0, 1))
